# Initial kernel scaffold; baseline (speedup 1.0000x reference)
import jax
import jax.numpy as jnp
from jax.experimental import pallas as pl


def kernel(x, edge_index, W, b):
    raise NotImplementedError("write your pallas kernel here")


# speedup vs baseline: 179.9031x; 179.9031x over previous
import functools

import jax
import jax.numpy as jnp
from jax import lax
from jax.experimental import pallas as pl
from jax.experimental.pallas import tpu as pltpu
from jax.experimental.pallas import tpu_sc as plsc

N = 50000
NP = 50176
E = 1600000
EP = 1638400
TILES = 32
EDGES_PER_TILE = EP // TILES
CHUNK = 2048
NCHUNKS = EDGES_PER_TILE // CHUNK
NPS = NP // 16

_F32 = jnp.float32
_I32 = jnp.int32


def _mesh():
    return plsc.VectorSubcoreMesh(core_axis_name="c", subcore_axis_name="s")


def _rsqrt_newton(d):
    bits = lax.bitcast_convert_type(d, _I32)
    seed = jnp.int32(0x5F3759DF) - lax.shift_right_arithmetic(bits, 1)
    y = lax.bitcast_convert_type(seed, _F32)
    for _ in range(3):
        y = y * (1.5 - 0.5 * d * y * y)
    return y


def _deg_call(dst1, zeros):
    @functools.partial(
        pl.kernel,
        out_type=(
            jax.ShapeDtypeStruct((NP,), _F32),
            jax.ShapeDtypeStruct((NP,), _F32),
        ),
        mesh=_mesh(),
        scratch_types=[
            pltpu.VMEM((CHUNK,), _I32),
            pltpu.VMEM((CHUNK,), _F32),
            pltpu.VMEM_SHARED((NP,), _F32),
        ],
    )
    def deg_kernel(dst_hbm, zeros_hbm, degp0_hbm, degp1_hbm, idx_v, ones_v, shared_deg):
        c = lax.axis_index("c")
        s = lax.axis_index("s")
        wid = c * 16 + s

        @pl.loop(0, CHUNK, step=16)
        def _(i):
            ones_v[pl.ds(i, 16)] = jnp.full((16,), 1.0, _F32)

        @pl.when(s == 0)
        def _():
            pltpu.sync_copy(zeros_hbm, shared_deg)

        plsc.subcore_barrier()

        base = wid * EDGES_PER_TILE

        @pl.loop(0, NCHUNKS)
        def _(j):
            pltpu.sync_copy(dst_hbm.at[pl.ds(base + j * CHUNK, CHUNK)], idx_v)
            pltpu.sync_copy(ones_v, shared_deg.at[idx_v], add=True)

        plsc.subcore_barrier()

        @pl.when(jnp.logical_and(s == 0, c == 0))
        def _():
            pltpu.sync_copy(shared_deg, degp0_hbm)

        @pl.when(jnp.logical_and(s == 0, c == 1))
        def _():
            pltpu.sync_copy(shared_deg, degp1_hbm)

    return deg_kernel(dst1, zeros)


def _scatter_call(src1, dst1, degp0, degp1, xp, zeros):
    @functools.partial(
        pl.kernel,
        out_type=(
            jax.ShapeDtypeStruct((NP,), _F32),
            jax.ShapeDtypeStruct((NP,), _F32),
            jax.ShapeDtypeStruct((NP,), _F32),
            jax.ShapeDtypeStruct((NP,), _F32),
        ),
        mesh=_mesh(),
        scratch_types=[
            pltpu.VMEM((CHUNK,), _I32),
            pltpu.VMEM((CHUNK,), _I32),
            pltpu.VMEM((CHUNK,), _F32),
            pltpu.VMEM((NPS,), _F32),
            pltpu.VMEM((NPS,), _F32),
            pltpu.VMEM((NPS,), _F32),
            pltpu.VMEM((NPS,), _F32),
            pltpu.VMEM((NPS,), _F32),
            pltpu.VMEM_SHARED((NP,), _F32),
            pltpu.VMEM_SHARED((NP,), _F32),
        ],
    )
    def scatter_kernel(src_hbm, dst_hbm, degp0_hbm, degp1_hbm, x_hbm, zeros_hbm,
                       sp0_hbm, sp1_hbm, t_hbm, dis_hbm,
                       src_v, dst_v, val_v, p0_v, p1_v, x_v, t_v, dis_v,
                       shared_t, shared_s):
        c = lax.axis_index("c")
        s = lax.axis_index("s")
        wid = c * 16 + s

        @pl.when(s == 0)
        def _():
            pltpu.sync_copy(zeros_hbm, shared_s)

        nb = s * NPS
        pltpu.sync_copy(degp0_hbm.at[pl.ds(nb, NPS)], p0_v)
        pltpu.sync_copy(degp1_hbm.at[pl.ds(nb, NPS)], p1_v)
        pltpu.sync_copy(x_hbm.at[pl.ds(nb, NPS)], x_v)

        @pl.loop(0, NPS, step=16)
        def _(i):
            sl = pl.ds(i, 16)
            d = p0_v[sl] + p1_v[sl] + 1.0
            y = _rsqrt_newton(d)
            dis_v[sl] = y
            t_v[sl] = x_v[sl] * y

        pltpu.sync_copy(t_v, shared_t.at[pl.ds(nb, NPS)])

        @pl.when(c == 0)
        def _():
            pltpu.sync_copy(t_v, t_hbm.at[pl.ds(nb, NPS)])
            pltpu.sync_copy(dis_v, dis_hbm.at[pl.ds(nb, NPS)])

        plsc.subcore_barrier()

        base = wid * EDGES_PER_TILE

        @pl.loop(0, NCHUNKS)
        def _(j):
            e0 = base + j * CHUNK
            pltpu.sync_copy(src_hbm.at[pl.ds(e0, CHUNK)], src_v)
            pltpu.sync_copy(dst_hbm.at[pl.ds(e0, CHUNK)], dst_v)
            pltpu.sync_copy(shared_t.at[src_v], val_v)
            pltpu.sync_copy(val_v, shared_s.at[dst_v], add=True)

        plsc.subcore_barrier()

        @pl.when(jnp.logical_and(s == 0, c == 0))
        def _():
            pltpu.sync_copy(shared_s, sp0_hbm)

        @pl.when(jnp.logical_and(s == 0, c == 1))
        def _():
            pltpu.sync_copy(shared_s, sp1_hbm)

    return scatter_kernel(src1, dst1, degp0, degp1, xp, zeros)


def _finish_body(W_ref, b_ref, sp0_ref, sp1_ref, t_ref, dis_ref, out_ref):
    u = (sp0_ref[...] + sp1_ref[...] + t_ref[...]) * dis_ref[...]
    for k in range(4):
        out_ref[k, :] = u * W_ref[0, k] + b_ref[k]


def _finish_call(sp0, sp1, t, dis, W, b):
    return pl.pallas_call(
        _finish_body,
        out_shape=jax.ShapeDtypeStruct((4, NP), _F32),
        in_specs=[
            pl.BlockSpec(memory_space=pltpu.SMEM),
            pl.BlockSpec(memory_space=pltpu.SMEM),
            pl.BlockSpec(memory_space=pltpu.VMEM),
            pl.BlockSpec(memory_space=pltpu.VMEM),
            pl.BlockSpec(memory_space=pltpu.VMEM),
            pl.BlockSpec(memory_space=pltpu.VMEM),
        ],
        out_specs=pl.BlockSpec(memory_space=pltpu.VMEM),
    )(W, b, sp0, sp1, t, dis)


def kernel(x, edge_index, W, b):
    src = edge_index[0].astype(_I32)
    dst = edge_index[1].astype(_I32)
    npad = EP - E
    pad_idx = (jnp.arange(npad, dtype=_I32) % (NP - N)) + N
    src1 = jnp.concatenate([src, pad_idx])
    dst1 = jnp.concatenate([dst, pad_idx])
    xp = jnp.pad(x[:, 0].astype(_F32), (0, NP - N))
    zeros = jnp.zeros((NP,), _F32)

    degp0, degp1 = _deg_call(dst1, zeros)
    sp0, sp1, t, dis = _scatter_call(src1, dst1, degp0, degp1, xp, zeros)
    out_t = _finish_call(sp0, sp1, t, dis, W.astype(_F32), b.astype(_F32))
    return out_t.T[:N, :]


# speedup vs baseline: 260.0155x; 1.4453x over previous
import functools

import jax
import jax.numpy as jnp
from jax import lax
from jax.experimental import pallas as pl
from jax.experimental.pallas import tpu as pltpu
from jax.experimental.pallas import tpu_sc as plsc

N = 50000
NP = 50176
E = 1600000
TILES = 32
EDGES_PER_TILE = E // TILES
CHUNK = 4096
NCHUNKS = EDGES_PER_TILE // CHUNK
TAIL = EDGES_PER_TILE - NCHUNKS * CHUNK
NBUF = 4
NPS = NP // 16

_F32 = jnp.float32
_I32 = jnp.int32


def _mesh():
    return plsc.VectorSubcoreMesh(core_axis_name="c", subcore_axis_name="s")


def _rsqrt_newton(d):
    bits = lax.bitcast_convert_type(d, _I32)
    seed = jnp.int32(0x5F3759DF) - lax.shift_right_arithmetic(bits, 1)
    y = lax.bitcast_convert_type(seed, _F32)
    for _ in range(3):
        y = y * (1.5 - 0.5 * d * y * y)
    return y


def _deg_call(dst1, zeros):
    @functools.partial(
        pl.kernel,
        out_type=(
            jax.ShapeDtypeStruct((NP,), _F32),
            jax.ShapeDtypeStruct((NP,), _F32),
        ),
        mesh=_mesh(),
        scratch_types=(
            [pltpu.VMEM((CHUNK,), _I32) for _ in range(NBUF)]
            + [pltpu.VMEM((CHUNK,), _F32)]
            + [pltpu.VMEM((TAIL,), _I32)]
            + [pltpu.SemaphoreType.DMA for _ in range(2 * NBUF)]
            + [pltpu.VMEM_SHARED((NP,), _F32)]
        ),
    )
    def deg_kernel(dst_hbm, zeros_hbm, degp0_hbm, degp1_hbm, *refs):
        dst_v = list(refs[0:NBUF])
        ones_v = refs[NBUF]
        tail_v = refs[NBUF + 1]
        isem = list(refs[NBUF + 2:NBUF + 2 + NBUF])
        ssem = list(refs[NBUF + 2 + NBUF:NBUF + 2 + 2 * NBUF])
        shared_deg = refs[-1]

        c = lax.axis_index("c")
        s = lax.axis_index("s")
        wid = c * 16 + s
        base = wid * EDGES_PER_TILE

        @pl.loop(0, CHUNK, step=16)
        def _(i):
            ones_v[pl.ds(i, 16)] = jnp.full((16,), 1.0, _F32)

        h_in = [None] * NBUF
        h_sc = [None] * NBUF
        for k in range(min(NBUF - 2, NCHUNKS)):
            h_in[k] = pltpu.async_copy(
                dst_hbm.at[pl.ds(base + k * CHUNK, CHUNK)], dst_v[k], isem[k])

        @pl.when(s == 0)
        def _():
            pltpu.sync_copy(zeros_hbm, shared_deg)

        plsc.subcore_barrier()

        for j in range(NCHUNKS):
            b = j % NBUF
            h_in[b].wait()
            h_sc[b] = pltpu.async_copy(
                ones_v, shared_deg.at[dst_v[b]], ssem[b], add=True)
            jn = j + (NBUF - 2)
            if jn < NCHUNKS:
                bb = jn % NBUF
                if h_sc[bb] is not None:
                    h_sc[bb].wait()
                    h_sc[bb] = None
                h_in[bb] = pltpu.async_copy(
                    dst_hbm.at[pl.ds(base + jn * CHUNK, CHUNK)], dst_v[bb],
                    isem[bb])
        for b in range(NBUF):
            if h_sc[b] is not None:
                h_sc[b].wait()

        pltpu.sync_copy(dst_hbm.at[pl.ds(base + NCHUNKS * CHUNK, TAIL)], tail_v)
        pltpu.sync_copy(ones_v.at[pl.ds(0, TAIL)], shared_deg.at[tail_v],
                        add=True)

        plsc.subcore_barrier()

        @pl.when(jnp.logical_and(s == 0, c == 0))
        def _():
            pltpu.sync_copy(shared_deg, degp0_hbm)

        @pl.when(jnp.logical_and(s == 0, c == 1))
        def _():
            pltpu.sync_copy(shared_deg, degp1_hbm)

    return deg_kernel(dst1, zeros)


def _scatter_call(src1, dst1, degp0, degp1, xp, zeros):
    @functools.partial(
        pl.kernel,
        out_type=(
            jax.ShapeDtypeStruct((NP,), _F32),
            jax.ShapeDtypeStruct((NP,), _F32),
            jax.ShapeDtypeStruct((NP,), _F32),
            jax.ShapeDtypeStruct((NP,), _F32),
        ),
        mesh=_mesh(),
        scratch_types=(
            [pltpu.VMEM((CHUNK,), _I32) for _ in range(NBUF)]
            + [pltpu.VMEM((CHUNK,), _I32) for _ in range(NBUF)]
            + [pltpu.VMEM((CHUNK,), _F32) for _ in range(NBUF)]
            + [pltpu.VMEM((TAIL,), _I32),
               pltpu.VMEM((TAIL,), _I32),
               pltpu.VMEM((TAIL,), _F32)]
            + [pltpu.VMEM((NPS,), _F32) for _ in range(5)]
            + [pltpu.SemaphoreType.DMA for _ in range(3 * NBUF + 1)]
            + [pltpu.VMEM_SHARED((NP,), _F32),
               pltpu.VMEM_SHARED((NP,), _F32)]
        ),
    )
    def scatter_kernel(src_hbm, dst_hbm, degp0_hbm, degp1_hbm, x_hbm,
                       zeros_hbm, sp0_hbm, sp1_hbm, t_hbm, dis_hbm, *refs):
        src_v = list(refs[0:NBUF])
        dst_v = list(refs[NBUF:2 * NBUF])
        val_v = list(refs[2 * NBUF:3 * NBUF])
        tsrc_v, tdst_v, tval_v = refs[3 * NBUF:3 * NBUF + 3]
        p0_v, p1_v, x_v, t_v, dis_v = refs[3 * NBUF + 3:3 * NBUF + 8]
        sems = list(refs[3 * NBUF + 8:3 * NBUF + 8 + 3 * NBUF + 1])
        isem = sems[0:NBUF]
        gsem = sems[NBUF:2 * NBUF]
        ssem = sems[2 * NBUF:3 * NBUF]
        psem = sems[3 * NBUF]
        shared_t, shared_s = refs[-2], refs[-1]

        c = lax.axis_index("c")
        s = lax.axis_index("s")
        wid = c * 16 + s
        base = wid * EDGES_PER_TILE
        nb = s * NPS

        h_in = [None] * NBUF
        h_g = [None] * NBUF
        h_sc = [None] * NBUF
        for k in range(min(NBUF - 2, NCHUNKS)):
            h_in[k] = (
                pltpu.async_copy(
                    src_hbm.at[pl.ds(base + k * CHUNK, CHUNK)], src_v[k],
                    isem[k]),
                pltpu.async_copy(
                    dst_hbm.at[pl.ds(base + k * CHUNK, CHUNK)], dst_v[k],
                    isem[k]),
            )

        hp0 = pltpu.async_copy(degp0_hbm.at[pl.ds(nb, NPS)], p0_v, psem)
        hp1 = pltpu.async_copy(degp1_hbm.at[pl.ds(nb, NPS)], p1_v, psem)
        hpx = pltpu.async_copy(x_hbm.at[pl.ds(nb, NPS)], x_v, psem)

        @pl.when(s == 0)
        def _():
            pltpu.sync_copy(zeros_hbm, shared_s)

        hp0.wait()
        hp1.wait()
        hpx.wait()

        @pl.loop(0, NPS, step=16)
        def _(i):
            sl = pl.ds(i, 16)
            d = p0_v[sl] + p1_v[sl] + 1.0
            y = _rsqrt_newton(d)
            dis_v[sl] = y
            t_v[sl] = x_v[sl] * y

        pltpu.sync_copy(t_v, shared_t.at[pl.ds(nb, NPS)])

        @pl.when(c == 0)
        def _():
            pltpu.sync_copy(t_v, t_hbm.at[pl.ds(nb, NPS)])
            pltpu.sync_copy(dis_v, dis_hbm.at[pl.ds(nb, NPS)])

        plsc.subcore_barrier()

        for j in range(NCHUNKS):
            b = j % NBUF
            h_in[b][0].wait()
            h_in[b][1].wait()
            h_g[b] = pltpu.async_copy(shared_t.at[src_v[b]], val_v[b], gsem[b])
            jn = j + (NBUF - 2)
            if jn < NCHUNKS:
                bb = jn % NBUF
                if h_sc[bb] is not None:
                    h_sc[bb].wait()
                    h_sc[bb] = None
                h_in[bb] = (
                    pltpu.async_copy(
                        src_hbm.at[pl.ds(base + jn * CHUNK, CHUNK)], src_v[bb],
                        isem[bb]),
                    pltpu.async_copy(
                        dst_hbm.at[pl.ds(base + jn * CHUNK, CHUNK)], dst_v[bb],
                        isem[bb]),
                )
            h_g[b].wait()
            h_sc[b] = pltpu.async_copy(
                val_v[b], shared_s.at[dst_v[b]], ssem[b], add=True)
        for b in range(NBUF):
            if h_sc[b] is not None:
                h_sc[b].wait()

        e0 = base + NCHUNKS * CHUNK
        pltpu.sync_copy(src_hbm.at[pl.ds(e0, TAIL)], tsrc_v)
        pltpu.sync_copy(dst_hbm.at[pl.ds(e0, TAIL)], tdst_v)
        pltpu.sync_copy(shared_t.at[tsrc_v], tval_v)
        pltpu.sync_copy(tval_v, shared_s.at[tdst_v], add=True)

        plsc.subcore_barrier()

        @pl.when(jnp.logical_and(s == 0, c == 0))
        def _():
            pltpu.sync_copy(shared_s, sp0_hbm)

        @pl.when(jnp.logical_and(s == 0, c == 1))
        def _():
            pltpu.sync_copy(shared_s, sp1_hbm)

    return scatter_kernel(src1, dst1, degp0, degp1, xp, zeros)


def _finish_body(W_ref, b_ref, sp0_ref, sp1_ref, t_ref, dis_ref, out_ref):
    u = (sp0_ref[...] + sp1_ref[...] + t_ref[...]) * dis_ref[...]
    for k in range(4):
        out_ref[k, :] = u * W_ref[0, k] + b_ref[k]


def _finish_call(sp0, sp1, t, dis, W, b):
    return pl.pallas_call(
        _finish_body,
        out_shape=jax.ShapeDtypeStruct((4, NP), _F32),
        in_specs=[
            pl.BlockSpec(memory_space=pltpu.SMEM),
            pl.BlockSpec(memory_space=pltpu.SMEM),
            pl.BlockSpec(memory_space=pltpu.VMEM),
            pl.BlockSpec(memory_space=pltpu.VMEM),
            pl.BlockSpec(memory_space=pltpu.VMEM),
            pl.BlockSpec(memory_space=pltpu.VMEM),
        ],
        out_specs=pl.BlockSpec(memory_space=pltpu.VMEM),
    )(W, b, sp0, sp1, t, dis)


def kernel(x, edge_index, W, b):
    src1 = edge_index[0].astype(_I32)
    dst1 = edge_index[1].astype(_I32)
    xp = jnp.pad(x[:, 0].astype(_F32), (0, NP - N))
    zeros = jnp.zeros((NP,), _F32)

    degp0, degp1 = _deg_call(dst1, zeros)
    sp0, sp1, t, dis = _scatter_call(src1, dst1, degp0, degp1, xp, zeros)
    out_t = _finish_call(sp0, sp1, t, dis, W.astype(_F32), b.astype(_F32))
    return out_t.T[:N, :]


# speedup vs baseline: 275.4734x; 1.0595x over previous
import dataclasses
import functools

import jax
import jax.numpy as jnp
from jax import lax
from jax.experimental import pallas as pl
from jax.experimental.pallas import tpu as pltpu
from jax.experimental.pallas import tpu_sc as plsc

N = 50000
NP = 50176
E = 1600000
TILES = 32
EDGES_PER_TILE = E // TILES
CHUNK = 4096
NCHUNKS = EDGES_PER_TILE // CHUNK
TAIL = EDGES_PER_TILE - NCHUNKS * CHUNK
NBUF = 4
NPS = NP // 16

_F32 = jnp.float32
_I32 = jnp.int32


def _mesh():
    return plsc.VectorSubcoreMesh(core_axis_name="c", subcore_axis_name="s")


def _sc_params():
    cp = pltpu.CompilerParams()
    if "needs_layout_passes" in pltpu.CompilerParams.__dataclass_fields__:
        cp = dataclasses.replace(cp, needs_layout_passes=False)
    return cp


def _rsqrt_newton(d):
    bits = lax.bitcast_convert_type(d, _I32)
    seed = jnp.int32(0x5F3759DF) - lax.shift_right_arithmetic(bits, 1)
    y = lax.bitcast_convert_type(seed, _F32)
    for _ in range(3):
        y = y * (1.5 - 0.5 * d * y * y)
    return y


def _deg_call(dst1, zeros):
    @functools.partial(
        pl.kernel,
        out_type=(
            jax.ShapeDtypeStruct((NP,), _F32),
            jax.ShapeDtypeStruct((NP,), _F32),
        ),
        mesh=_mesh(),
        scratch_types=(
            [pltpu.VMEM((CHUNK,), _I32) for _ in range(NBUF)]
            + [pltpu.VMEM((CHUNK,), _F32)]
            + [pltpu.VMEM((TAIL,), _I32)]
            + [pltpu.SemaphoreType.DMA for _ in range(2 * NBUF)]
            + [pltpu.VMEM_SHARED((NP,), _F32)]
        ),
    )
    def deg_kernel(dst_hbm, zeros_hbm, degp0_hbm, degp1_hbm, *refs):
        dst_v = list(refs[0:NBUF])
        ones_v = refs[NBUF]
        tail_v = refs[NBUF + 1]
        isem = list(refs[NBUF + 2:NBUF + 2 + NBUF])
        ssem = list(refs[NBUF + 2 + NBUF:NBUF + 2 + 2 * NBUF])
        shared_deg = refs[-1]

        c = lax.axis_index("c")
        s = lax.axis_index("s")
        wid = c * 16 + s
        base = wid * EDGES_PER_TILE

        @pl.loop(0, CHUNK, step=16)
        def _(i):
            ones_v[pl.ds(i, 16)] = jnp.full((16,), 1.0, _F32)

        h_in = [None] * NBUF
        h_sc = [None] * NBUF
        for k in range(min(NBUF - 2, NCHUNKS)):
            h_in[k] = pltpu.async_copy(
                dst_hbm.at[pl.ds(base + k * CHUNK, CHUNK)], dst_v[k], isem[k])

        @pl.when(s == 0)
        def _():
            pltpu.sync_copy(zeros_hbm, shared_deg)

        plsc.subcore_barrier()

        for j in range(NCHUNKS):
            b = j % NBUF
            h_in[b].wait()
            h_sc[b] = pltpu.async_copy(
                ones_v, shared_deg.at[dst_v[b]], ssem[b], add=True)
            jn = j + (NBUF - 2)
            if jn < NCHUNKS:
                bb = jn % NBUF
                if h_sc[bb] is not None:
                    h_sc[bb].wait()
                    h_sc[bb] = None
                h_in[bb] = pltpu.async_copy(
                    dst_hbm.at[pl.ds(base + jn * CHUNK, CHUNK)], dst_v[bb],
                    isem[bb])
        for b in range(NBUF):
            if h_sc[b] is not None:
                h_sc[b].wait()

        pltpu.sync_copy(dst_hbm.at[pl.ds(base + NCHUNKS * CHUNK, TAIL)], tail_v)
        pltpu.sync_copy(ones_v.at[pl.ds(0, TAIL)], shared_deg.at[tail_v],
                        add=True)

        plsc.subcore_barrier()

        @pl.when(jnp.logical_and(s == 0, c == 0))
        def _():
            pltpu.sync_copy(shared_deg, degp0_hbm)

        @pl.when(jnp.logical_and(s == 0, c == 1))
        def _():
            pltpu.sync_copy(shared_deg, degp1_hbm)

    return deg_kernel(dst1, zeros)


def _scatter_call(src1, dst1, degp0, degp1, xp, zeros):
    @functools.partial(
        pl.kernel,
        out_type=(
            jax.ShapeDtypeStruct((NP,), _F32),
            jax.ShapeDtypeStruct((NP,), _F32),
            jax.ShapeDtypeStruct((NP,), _F32),
            jax.ShapeDtypeStruct((NP,), _F32),
        ),
        mesh=_mesh(),
        compiler_params=_sc_params(),
        scratch_types=(
            [pltpu.VMEM((CHUNK,), _I32) for _ in range(NBUF)]
            + [pltpu.VMEM((CHUNK,), _I32) for _ in range(NBUF)]
            + [pltpu.VMEM((CHUNK,), _F32) for _ in range(NBUF)]
            + [pltpu.VMEM((TAIL,), _I32),
               pltpu.VMEM((TAIL,), _I32),
               pltpu.VMEM((TAIL,), _F32)]
            + [pltpu.VMEM((NPS,), _F32) for _ in range(5)]
            + [pltpu.VMEM((NP,), _F32)]
            + [pltpu.SemaphoreType.DMA for _ in range(3 * NBUF + 1)]
            + [pltpu.VMEM_SHARED((NP,), _F32),
               pltpu.VMEM_SHARED((NP,), _F32)]
        ),
    )
    def scatter_kernel(src_hbm, dst_hbm, degp0_hbm, degp1_hbm, x_hbm,
                       zeros_hbm, sp0_hbm, sp1_hbm, t_hbm, dis_hbm, *refs):
        src_v = list(refs[0:NBUF])
        dst_v = list(refs[NBUF:2 * NBUF])
        val_v = list(refs[2 * NBUF:3 * NBUF])
        tsrc_v, tdst_v, tval_v = refs[3 * NBUF:3 * NBUF + 3]
        p0_v, p1_v, x_v, t_v, dis_v = refs[3 * NBUF + 3:3 * NBUF + 8]
        t_full = refs[3 * NBUF + 8]
        sems = list(refs[3 * NBUF + 9:3 * NBUF + 9 + 3 * NBUF + 1])
        isem = sems[0:NBUF]
        gsem = sems[NBUF:2 * NBUF]
        ssem = sems[2 * NBUF:3 * NBUF]
        psem = sems[3 * NBUF]
        shared_t, shared_s = refs[-2], refs[-1]

        c = lax.axis_index("c")
        s = lax.axis_index("s")
        wid = c * 16 + s
        base = wid * EDGES_PER_TILE
        nb = s * NPS

        h_in = [None] * NBUF
        h_g = [None] * NBUF
        h_sc = [None] * NBUF
        for k in range(min(NBUF - 2, NCHUNKS)):
            h_in[k] = (
                pltpu.async_copy(
                    src_hbm.at[pl.ds(base + k * CHUNK, CHUNK)], src_v[k],
                    isem[k]),
                pltpu.async_copy(
                    dst_hbm.at[pl.ds(base + k * CHUNK, CHUNK)], dst_v[k],
                    isem[k]),
            )

        hp0 = pltpu.async_copy(degp0_hbm.at[pl.ds(nb, NPS)], p0_v, psem)
        hp1 = pltpu.async_copy(degp1_hbm.at[pl.ds(nb, NPS)], p1_v, psem)
        hpx = pltpu.async_copy(x_hbm.at[pl.ds(nb, NPS)], x_v, psem)

        @pl.when(s == 0)
        def _():
            pltpu.sync_copy(zeros_hbm, shared_s)

        hp0.wait()
        hp1.wait()
        hpx.wait()

        @pl.loop(0, NPS, step=16)
        def _(i):
            sl = pl.ds(i, 16)
            d = p0_v[sl] + p1_v[sl] + 1.0
            y = _rsqrt_newton(d)
            dis_v[sl] = y
            t_v[sl] = x_v[sl] * y

        pltpu.sync_copy(t_v, shared_t.at[pl.ds(nb, NPS)])

        @pl.when(c == 0)
        def _():
            pltpu.sync_copy(t_v, t_hbm.at[pl.ds(nb, NPS)])
            pltpu.sync_copy(dis_v, dis_hbm.at[pl.ds(nb, NPS)])

        plsc.subcore_barrier()

        pltpu.sync_copy(shared_t, t_full)

        for j in range(NCHUNKS):
            b = j % NBUF
            h_in[b][0].wait()
            h_in[b][1].wait()
            if h_sc[b] is not None:
                h_sc[b].wait()
                h_sc[b] = None

            @pl.loop(0, CHUNK, step=16)
            def _(i, _b=b):
                sl = pl.ds(i, 16)
                val_v[_b][sl] = plsc.load_gather(t_full, [src_v[_b][sl]])

            h_sc[b] = pltpu.async_copy(
                val_v[b], shared_s.at[dst_v[b]], ssem[b], add=True)
            jn = j + (NBUF - 2)
            if jn < NCHUNKS:
                bb = jn % NBUF
                if h_sc[bb] is not None:
                    h_sc[bb].wait()
                    h_sc[bb] = None
                h_in[bb] = (
                    pltpu.async_copy(
                        src_hbm.at[pl.ds(base + jn * CHUNK, CHUNK)], src_v[bb],
                        isem[bb]),
                    pltpu.async_copy(
                        dst_hbm.at[pl.ds(base + jn * CHUNK, CHUNK)], dst_v[bb],
                        isem[bb]),
                )
        for b in range(NBUF):
            if h_sc[b] is not None:
                h_sc[b].wait()

        e0 = base + NCHUNKS * CHUNK
        pltpu.sync_copy(src_hbm.at[pl.ds(e0, TAIL)], tsrc_v)
        pltpu.sync_copy(dst_hbm.at[pl.ds(e0, TAIL)], tdst_v)

        @pl.loop(0, TAIL, step=16)
        def _(i):
            sl = pl.ds(i, 16)
            tval_v[sl] = plsc.load_gather(t_full, [tsrc_v[sl]])

        pltpu.sync_copy(tval_v, shared_s.at[tdst_v], add=True)

        plsc.subcore_barrier()

        @pl.when(jnp.logical_and(s == 0, c == 0))
        def _():
            pltpu.sync_copy(shared_s, sp0_hbm)

        @pl.when(jnp.logical_and(s == 0, c == 1))
        def _():
            pltpu.sync_copy(shared_s, sp1_hbm)

    return scatter_kernel(src1, dst1, degp0, degp1, xp, zeros)


def _finish_body(W_ref, b_ref, sp0_ref, sp1_ref, t_ref, dis_ref, out_ref):
    u = (sp0_ref[...] + sp1_ref[...] + t_ref[...]) * dis_ref[...]
    for k in range(4):
        out_ref[k, :] = u * W_ref[0, k] + b_ref[k]


def _finish_call(sp0, sp1, t, dis, W, b):
    return pl.pallas_call(
        _finish_body,
        out_shape=jax.ShapeDtypeStruct((4, NP), _F32),
        in_specs=[
            pl.BlockSpec(memory_space=pltpu.SMEM),
            pl.BlockSpec(memory_space=pltpu.SMEM),
            pl.BlockSpec(memory_space=pltpu.VMEM),
            pl.BlockSpec(memory_space=pltpu.VMEM),
            pl.BlockSpec(memory_space=pltpu.VMEM),
            pl.BlockSpec(memory_space=pltpu.VMEM),
        ],
        out_specs=pl.BlockSpec(memory_space=pltpu.VMEM),
    )(W, b, sp0, sp1, t, dis)


def kernel(x, edge_index, W, b):
    dst1 = lax.optimization_barrier(edge_index[1].astype(_I32))
    src1 = lax.optimization_barrier(edge_index[0].astype(_I32))
    xp = jnp.pad(x[:, 0].astype(_F32), (0, NP - N))
    zeros = jnp.zeros((NP,), _F32)

    degp0, degp1 = _deg_call(dst1, zeros)
    sp0, sp1, t, dis = _scatter_call(src1, dst1, degp0, degp1, xp, zeros)
    out_t = _finish_call(sp0, sp1, t, dis, W.astype(_F32), b.astype(_F32))
    return out_t.T[:N, :]


# speedup vs baseline: 338.0814x; 1.2273x over previous
import dataclasses
import functools

import jax
import jax.numpy as jnp
from jax import lax
from jax.experimental import pallas as pl
from jax.experimental.pallas import tpu as pltpu
from jax.experimental.pallas import tpu_sc as plsc

N = 50000
NP = 50176
E = 1600000
TILES = 32
CHUNK = 4096
NCHUNKS = 12
SMALL = 49920
BIG_TILES = 20
TAIL_BIG = 896
TAIL_SMALL = 768
NBUF = 3
BNBUF = 2
NPS = NP // 16

_F32 = jnp.float32
_I32 = jnp.int32


def _mesh():
    return plsc.VectorSubcoreMesh(core_axis_name="c", subcore_axis_name="s")


def _sc_params():
    cp = pltpu.CompilerParams()
    if "needs_layout_passes" in pltpu.CompilerParams.__dataclass_fields__:
        cp = dataclasses.replace(cp, needs_layout_passes=False)
    return cp


def _edge_base(wid):
    return wid * SMALL + jnp.minimum(wid, BIG_TILES) * 128


def _rsqrt_newton(d):
    bits = lax.bitcast_convert_type(d, _I32)
    seed = jnp.int32(0x5F3759DF) - lax.shift_right_arithmetic(bits, 1)
    y = lax.bitcast_convert_type(seed, _F32)
    for _ in range(3):
        y = y * (1.5 - 0.5 * d * y * y)
    return y


def _deg_call(ei, zeros):
    @functools.partial(
        pl.kernel,
        out_type=(
            jax.ShapeDtypeStruct((NP,), _F32),
            jax.ShapeDtypeStruct((NP,), _F32),
        ),
        mesh=_mesh(),
        scratch_types=(
            [pltpu.VMEM((2, CHUNK), _I32) for _ in range(NBUF)]
            + [pltpu.VMEM((CHUNK,), _I32) for _ in range(NBUF)]
            + [pltpu.VMEM((CHUNK,), _F32)]
            + [pltpu.VMEM((2, TAIL_BIG), _I32)]
            + [pltpu.VMEM((2, TAIL_SMALL), _I32)]
            + [pltpu.VMEM((TAIL_BIG,), _I32)]
            + [pltpu.VMEM((TAIL_SMALL,), _I32)]
            + [pltpu.SemaphoreType.DMA for _ in range(2 * NBUF)]
            + [pltpu.VMEM_SHARED((NP,), _F32)]
            + [pltpu.VMEM_SHARED((16 * CHUNK,), _I32)]
        ),
    )
    def deg_kernel(ei_hbm, zeros_hbm, degp0_hbm, degp1_hbm, *refs):
        edg_v = list(refs[0:NBUF])
        dstc_v = list(refs[NBUF:2 * NBUF])
        ones_v = refs[2 * NBUF]
        tbig_v = refs[2 * NBUF + 1]
        tsmall_v = refs[2 * NBUF + 2]
        tbd_v = refs[2 * NBUF + 3]
        tsd_v = refs[2 * NBUF + 4]
        isem = list(refs[2 * NBUF + 5:2 * NBUF + 5 + NBUF])
        ssem = list(refs[2 * NBUF + 5 + NBUF:2 * NBUF + 5 + 2 * NBUF])
        shared_deg = refs[-2]
        stage_d = refs[-1]

        c = lax.axis_index("c")
        s = lax.axis_index("s")
        wid = c * 16 + s
        base = _edge_base(wid)

        @pl.loop(0, CHUNK, step=16)
        def _(i):
            ones_v[pl.ds(i, 16)] = jnp.full((16,), 1.0, _F32)

        h_in = [None] * NBUF
        h_sc = [None] * NBUF
        for k in range(min(NBUF - 1, NCHUNKS)):
            h_in[k] = pltpu.async_copy(
                ei_hbm.at[:, pl.ds(base + k * CHUNK, CHUNK)], edg_v[k],
                isem[k])

        @pl.when(s == 0)
        def _():
            pltpu.sync_copy(zeros_hbm, shared_deg)

        plsc.subcore_barrier()

        for j in range(NCHUNKS):
            b = j % NBUF
            h_in[b].wait()
            if h_sc[b] is not None:
                h_sc[b].wait()
                h_sc[b] = None
            st = s * CHUNK
            pltpu.sync_copy(edg_v[b].at[1], stage_d.at[pl.ds(st, CHUNK)])
            pltpu.sync_copy(stage_d.at[pl.ds(st, CHUNK)], dstc_v[b])
            h_sc[b] = pltpu.async_copy(
                ones_v, shared_deg.at[dstc_v[b]], ssem[b], add=True)
            jn = j + (NBUF - 1)
            if jn < NCHUNKS:
                bb = jn % NBUF
                h_in[bb] = pltpu.async_copy(
                    ei_hbm.at[:, pl.ds(base + jn * CHUNK, CHUNK)], edg_v[bb],
                    isem[bb])
        for b in range(NBUF):
            if h_sc[b] is not None:
                h_sc[b].wait()

        t0 = base + NCHUNKS * CHUNK

        @pl.when(wid < BIG_TILES)
        def _():
            pltpu.sync_copy(ei_hbm.at[:, pl.ds(t0, TAIL_BIG)], tbig_v)

            pltpu.sync_copy(tbig_v.at[1],
                            stage_d.at[pl.ds(s * CHUNK, TAIL_BIG)])
            pltpu.sync_copy(stage_d.at[pl.ds(s * CHUNK, TAIL_BIG)], tbd_v)
            pltpu.sync_copy(ones_v.at[pl.ds(0, TAIL_BIG)],
                            shared_deg.at[tbd_v], add=True)

        @pl.when(wid >= BIG_TILES)
        def _():
            pltpu.sync_copy(ei_hbm.at[:, pl.ds(t0, TAIL_SMALL)], tsmall_v)

            pltpu.sync_copy(tsmall_v.at[1],
                            stage_d.at[pl.ds(s * CHUNK, TAIL_SMALL)])
            pltpu.sync_copy(stage_d.at[pl.ds(s * CHUNK, TAIL_SMALL)], tsd_v)
            pltpu.sync_copy(ones_v.at[pl.ds(0, TAIL_SMALL)],
                            shared_deg.at[tsd_v], add=True)

        plsc.subcore_barrier()

        @pl.when(jnp.logical_and(s == 0, c == 0))
        def _():
            pltpu.sync_copy(shared_deg, degp0_hbm)

        @pl.when(jnp.logical_and(s == 0, c == 1))
        def _():
            pltpu.sync_copy(shared_deg, degp1_hbm)

    return deg_kernel(ei, zeros)


def _scatter_call(ei, degp0, degp1, xp, zeros):
    @functools.partial(
        pl.kernel,
        out_type=(
            jax.ShapeDtypeStruct((NP,), _F32),
            jax.ShapeDtypeStruct((NP,), _F32),
            jax.ShapeDtypeStruct((NP,), _F32),
            jax.ShapeDtypeStruct((NP,), _F32),
        ),
        mesh=_mesh(),
        compiler_params=_sc_params(),
        scratch_types=(
            [pltpu.VMEM((2, CHUNK), _I32) for _ in range(BNBUF)]
            + [pltpu.VMEM((CHUNK,), _F32) for _ in range(BNBUF)]
            + [pltpu.VMEM((CHUNK,), _I32) for _ in range(BNBUF)]
            + [pltpu.VMEM((CHUNK,), _I32) for _ in range(BNBUF)]
            + [pltpu.VMEM((2, TAIL_BIG), _I32),
               pltpu.VMEM((2, TAIL_SMALL), _I32),
               pltpu.VMEM((TAIL_BIG,), _F32),
               pltpu.VMEM((TAIL_SMALL,), _F32),
               pltpu.VMEM((TAIL_BIG,), _I32),
               pltpu.VMEM((TAIL_SMALL,), _I32),
               pltpu.VMEM((TAIL_BIG,), _I32),
               pltpu.VMEM((TAIL_SMALL,), _I32)]
            + [pltpu.VMEM((NPS,), _F32) for _ in range(5)]
            + [pltpu.VMEM((NP,), _F32)]
            + [pltpu.SemaphoreType.DMA for _ in range(4 * BNBUF + 1)]
            + [pltpu.VMEM_SHARED((NP,), _F32),
               pltpu.VMEM_SHARED((NP,), _F32)]
            + [pltpu.VMEM_SHARED((32 * CHUNK,), _I32)]
        ),
    )
    def scatter_kernel(ei_hbm, degp0_hbm, degp1_hbm, x_hbm,
                       zeros_hbm, sp0_hbm, sp1_hbm, t_hbm, dis_hbm, *refs):
        edg_v = list(refs[0:BNBUF])
        val_v = list(refs[BNBUF:2 * BNBUF])
        dstc_v = list(refs[2 * BNBUF:3 * BNBUF])
        srcc_v = list(refs[3 * BNBUF:4 * BNBUF])
        (tbig_v, tsmall_v, tbval_v, tsval_v,
         tbd_v, tsd_v, tbs_v, tss_v) = refs[4 * BNBUF:4 * BNBUF + 8]
        p0_v, p1_v, x_v, t_v, dis_v = refs[4 * BNBUF + 8:4 * BNBUF + 13]
        t_full = refs[4 * BNBUF + 13]
        sems = list(refs[4 * BNBUF + 14:4 * BNBUF + 14 + 4 * BNBUF + 1])
        isem = sems[0:BNBUF]
        esem = sems[BNBUF:2 * BNBUF]
        ssem = sems[2 * BNBUF:3 * BNBUF]
        psem = sems[4 * BNBUF]
        shared_t, shared_s = refs[-3], refs[-2]
        stage_b = refs[-1]

        c = lax.axis_index("c")
        s = lax.axis_index("s")
        wid = c * 16 + s
        base = _edge_base(wid)
        nb = s * NPS

        h_in = [None] * BNBUF
        h_sc = [None] * BNBUF
        for k in range(min(BNBUF, NCHUNKS)):
            h_in[k] = pltpu.async_copy(
                ei_hbm.at[:, pl.ds(base + k * CHUNK, CHUNK)], edg_v[k],
                isem[k])

        hp0 = pltpu.async_copy(degp0_hbm.at[pl.ds(nb, NPS)], p0_v, psem)
        hp1 = pltpu.async_copy(degp1_hbm.at[pl.ds(nb, NPS)], p1_v, psem)
        hpx = pltpu.async_copy(x_hbm.at[pl.ds(nb, NPS)], x_v, psem)

        @pl.when(s == 0)
        def _():
            pltpu.sync_copy(zeros_hbm, shared_s)

        hp0.wait()
        hp1.wait()
        hpx.wait()

        @pl.loop(0, NPS, step=16)
        def _(i):
            sl = pl.ds(i, 16)
            d = p0_v[sl] + p1_v[sl] + 1.0
            y = _rsqrt_newton(d)
            dis_v[sl] = y
            t_v[sl] = x_v[sl] * y

        pltpu.sync_copy(t_v, shared_t.at[pl.ds(nb, NPS)])

        @pl.when(c == 0)
        def _():
            pltpu.sync_copy(t_v, t_hbm.at[pl.ds(nb, NPS)])
            pltpu.sync_copy(dis_v, dis_hbm.at[pl.ds(nb, NPS)])

        plsc.subcore_barrier()

        pltpu.sync_copy(shared_t, t_full)

        for j in range(NCHUNKS):
            b = j % BNBUF
            h_in[b].wait()
            if h_sc[b] is not None:
                h_sc[b].wait()
                h_sc[b] = None
            st = s * 2 * CHUNK
            pltpu.sync_copy(edg_v[b].at[0], stage_b.at[pl.ds(st, CHUNK)])
            pltpu.sync_copy(edg_v[b].at[1],
                            stage_b.at[pl.ds(st + CHUNK, CHUNK)])
            pltpu.sync_copy(stage_b.at[pl.ds(st, CHUNK)], srcc_v[b])
            pltpu.sync_copy(stage_b.at[pl.ds(st + CHUNK, CHUNK)], dstc_v[b])
            jn = j + BNBUF
            if jn < NCHUNKS:
                h_in[b] = pltpu.async_copy(
                    ei_hbm.at[:, pl.ds(base + jn * CHUNK, CHUNK)], edg_v[b],
                    isem[b])

            @pl.loop(0, CHUNK, step=16)
            def _(i, _b=b):
                sl = pl.ds(i, 16)
                val_v[_b][sl] = plsc.load_gather(t_full, [srcc_v[_b][sl]])

            h_sc[b] = pltpu.async_copy(
                val_v[b], shared_s.at[dstc_v[b]], ssem[b], add=True)
        for b in range(BNBUF):
            if h_sc[b] is not None:
                h_sc[b].wait()

        t0 = base + NCHUNKS * CHUNK

        @pl.when(wid < BIG_TILES)
        def _():
            pltpu.sync_copy(ei_hbm.at[:, pl.ds(t0, TAIL_BIG)], tbig_v)

            st2 = s * 2 * CHUNK
            pltpu.sync_copy(tbig_v.at[0], stage_b.at[pl.ds(st2, TAIL_BIG)])
            pltpu.sync_copy(tbig_v.at[1],
                            stage_b.at[pl.ds(st2 + CHUNK, TAIL_BIG)])
            pltpu.sync_copy(stage_b.at[pl.ds(st2, TAIL_BIG)], tbs_v)
            pltpu.sync_copy(stage_b.at[pl.ds(st2 + CHUNK, TAIL_BIG)], tbd_v)

            @pl.loop(0, TAIL_BIG, step=16)
            def _(i):
                sl = pl.ds(i, 16)
                tbval_v[sl] = plsc.load_gather(t_full, [tbs_v[sl]])

            pltpu.sync_copy(tbval_v, shared_s.at[tbd_v], add=True)

        @pl.when(wid >= BIG_TILES)
        def _():
            pltpu.sync_copy(ei_hbm.at[:, pl.ds(t0, TAIL_SMALL)], tsmall_v)

            st2 = s * 2 * CHUNK
            pltpu.sync_copy(tsmall_v.at[0], stage_b.at[pl.ds(st2, TAIL_SMALL)])
            pltpu.sync_copy(tsmall_v.at[1],
                            stage_b.at[pl.ds(st2 + CHUNK, TAIL_SMALL)])
            pltpu.sync_copy(stage_b.at[pl.ds(st2, TAIL_SMALL)], tss_v)
            pltpu.sync_copy(stage_b.at[pl.ds(st2 + CHUNK, TAIL_SMALL)], tsd_v)

            @pl.loop(0, TAIL_SMALL, step=16)
            def _(i):
                sl = pl.ds(i, 16)
                tsval_v[sl] = plsc.load_gather(t_full, [tss_v[sl]])

            pltpu.sync_copy(tsval_v, shared_s.at[tsd_v], add=True)

        plsc.subcore_barrier()

        @pl.when(jnp.logical_and(s == 0, c == 0))
        def _():
            pltpu.sync_copy(shared_s, sp0_hbm)

        @pl.when(jnp.logical_and(s == 0, c == 1))
        def _():
            pltpu.sync_copy(shared_s, sp1_hbm)

    return scatter_kernel(ei, degp0, degp1, xp, zeros)


def _finish_body(W_ref, b_ref, sp0_ref, sp1_ref, t_ref, dis_ref, out_ref):
    u = (sp0_ref[...] + sp1_ref[...] + t_ref[...]) * dis_ref[...]
    for k in range(4):
        out_ref[k, :] = u * W_ref[0, k] + b_ref[k]


def _finish_call(sp0, sp1, t, dis, W, b):
    return pl.pallas_call(
        _finish_body,
        out_shape=jax.ShapeDtypeStruct((4, NP), _F32),
        in_specs=[
            pl.BlockSpec(memory_space=pltpu.SMEM),
            pl.BlockSpec(memory_space=pltpu.SMEM),
            pl.BlockSpec(memory_space=pltpu.VMEM),
            pl.BlockSpec(memory_space=pltpu.VMEM),
            pl.BlockSpec(memory_space=pltpu.VMEM),
            pl.BlockSpec(memory_space=pltpu.VMEM),
        ],
        out_specs=pl.BlockSpec(memory_space=pltpu.VMEM),
    )(W, b, sp0, sp1, t, dis)


def kernel(x, edge_index, W, b):
    ei = edge_index.astype(_I32)
    xp = jnp.pad(x[:, 0].astype(_F32), (0, NP - N))
    zeros = jnp.zeros((NP,), _F32)

    degp0, degp1 = _deg_call(ei, zeros)
    sp0, sp1, t, dis = _scatter_call(ei, degp0, degp1, xp, zeros)
    out_t = _finish_call(sp0, sp1, t, dis, W.astype(_F32), b.astype(_F32))
    return out_t.T[:N, :]
